# baseline (speedup 1.0000x reference)
import functools

import jax
import jax.numpy as jnp
from jax import lax
from jax.experimental import pallas as pl
from jax.experimental.pallas import tpu as pltpu
from jax.experimental.pallas import tpu_sc as plsc


def _make_sc_embed(D, B, NC, NS, L):
    NW = NC * NS
    TPW = B // NW
    HTPW = TPW // 2
    QTP = HTPW // 2
    FIRE = 16
    D2 = 2 * D

    mesh = plsc.VectorSubcoreMesh(core_axis_name="c", subcore_axis_name="s")

    @functools.partial(
        pl.kernel,
        mesh=mesh,
        out_type=jax.ShapeDtypeStruct((B, D), jnp.float32),
        scratch_types=[
            pltpu.VMEM((TPW,), jnp.int32),
            pltpu.VMEM((HTPW, D2), jnp.float32),
            pltpu.VMEM((QTP, D), jnp.float32),
            pltpu.VMEM((L,), jnp.float32),
            pltpu.SemaphoreType.DMA,
            pltpu.SemaphoreType.DMA,
        ],
    )
    def sc_embed(tok_hbm, scale_hbm, table_hbm, out_hbm,
                 idx_v, pairs, stage, scale_v, gsem, ssem):
        wid = lax.axis_index("s") * NC + lax.axis_index("c")
        pltpu.sync_copy(tok_hbm.at[wid], idx_v)
        pltpu.sync_copy(scale_hbm, scale_v)
        s = scale_v[...]

        for p in range(2):
            def fire_body(g, carry, p=p):
                idx16 = idx_v[pl.ds(p * HTPW + g * FIRE, FIRE)]
                p16 = idx16 >> 1
                for k in range(FIRE):
                    pltpu.async_copy(
                        table_hbm.at[p16[k]], pairs.at[g * FIRE + k], gsem)
                return carry

            lax.fori_loop(0, HTPW // FIRE, fire_body, 0)
            pltpu.make_async_copy(
                table_hbm.at[pl.ds(0, HTPW)], pairs, gsem).wait()

            for h2 in range(2):
                def g_body(g, carry, p=p, h2=h2):
                    base = h2 * QTP + g * L
                    idx16 = idx_v[pl.ds(p * HTPW + base, L)]
                    h16 = (idx16 & 1) * D
                    for l in range(L):
                        r = base + l
                        lr = g * L + l
                        h = h16[l]
                        for j in range(D // L):
                            v = pairs[r, pl.ds(h + j * L, L)]
                            stage[lr, pl.ds(j * L, L)] = v * s
                    return carry

                lax.fori_loop(0, QTP // L, g_body, 0)
                pltpu.async_copy(
                    stage,
                    out_hbm.at[pl.ds(wid * TPW + p * HTPW + h2 * QTP, QTP)],
                    ssem).wait()

    return sc_embed


def kernel(token_ids, embed_weight, scale):
    B0, B1 = token_ids.shape
    V, D = embed_weight.shape
    B = B0 * B1
    info = plsc.get_sparse_core_info()
    NC, NS, L = info.num_cores, info.num_subcores, info.num_lanes
    NW = NC * NS

    tok = token_ids.reshape(NW, B // NW).astype(jnp.int32)
    scale16 = jnp.broadcast_to(scale.astype(jnp.float32).reshape(1), (L,))
    table2 = jnp.concatenate([embed_weight[0::2], embed_weight[1::2]], axis=1)
    out = _make_sc_embed(D, B, NC, NS, L)(tok, scale16, table2)
    return out.reshape(B0, B1, D)


# speedup vs baseline: 23.1653x; 23.1653x over previous
import functools

import jax
import jax.numpy as jnp
from jax import lax
from jax.experimental import pallas as pl
from jax.experimental.pallas import tpu as pltpu
from jax.experimental.pallas import tpu_sc as plsc


def _make_sc_embed(D, B, NC, NS, L):
    NW = NC * NS
    TPW = B // NW
    HTPW = TPW // 2
    FIRE = 16

    mesh = plsc.VectorSubcoreMesh(core_axis_name="c", subcore_axis_name="s")

    @functools.partial(
        pl.kernel,
        mesh=mesh,
        out_type=jax.ShapeDtypeStruct((B, D), jnp.float32),
        scratch_types=[
            pltpu.VMEM((TPW,), jnp.int32),
            pltpu.VMEM((HTPW, D), jnp.float32),
            pltpu.VMEM((L,), jnp.float32),
            pltpu.SemaphoreType.DMA,
            pltpu.SemaphoreType.DMA,
        ],
    )
    def sc_embed(tok_hbm, scale_hbm, table_hbm, out_hbm,
                 idx_v, stage, scale_v, gsem, ssem):
        wid = lax.axis_index("s") * NC + lax.axis_index("c")
        pltpu.sync_copy(tok_hbm.at[wid], idx_v)
        pltpu.sync_copy(scale_hbm, scale_v)
        s = scale_v[...]

        for p in range(2):
            def fire_body(g, carry, p=p):
                idx16 = idx_v[pl.ds(p * HTPW + g * FIRE, FIRE)]
                for k in range(FIRE):
                    lg = g * FIRE + k
                    i_t = idx16[k]
                    pltpu.async_copy(
                        table_hbm.at[i_t], stage.at[lg], gsem)
                return carry

            lax.fori_loop(0, HTPW // FIRE, fire_body, 0)
            pltpu.make_async_copy(
                table_hbm.at[pl.ds(0, HTPW)], stage, gsem).wait()

            def scale_body(r, carry):
                for k in range(D // L):
                    sl = (r, pl.ds(k * L, L))
                    stage[sl] = stage[sl] * s
                return carry

            lax.fori_loop(0, HTPW, scale_body, 0)
            pltpu.async_copy(
                stage,
                out_hbm.at[pl.ds(wid * TPW + p * HTPW, HTPW)],
                ssem).wait()

    return sc_embed


def kernel(token_ids, embed_weight, scale):
    B0, B1 = token_ids.shape
    V, D = embed_weight.shape
    B = B0 * B1
    info = plsc.get_sparse_core_info()
    NC, NS, L = info.num_cores, info.num_subcores, info.num_lanes
    NW = NC * NS

    tok = token_ids.reshape(NW, B // NW).astype(jnp.int32)
    scale16 = jnp.broadcast_to(scale.astype(jnp.float32).reshape(1), (L,))
    out = _make_sc_embed(D, B, NC, NS, L)(tok, scale16, embed_weight)
    return out.reshape(B0, B1, D)
